# baseline (speedup 1.0000x reference)
import dataclasses
import functools

import jax
import jax.numpy as jnp
from jax.experimental import pallas as pl
from jax.experimental.pallas import tpu as pltpu
from jax.experimental.pallas import tpu_sc as plsc

LANES = 16
ROWS_PER_BLOCK = 4


def kernel(x, permutation):
    batch, dim = x.shape
    perm = permutation.astype(jnp.int32)
    mesh = plsc.VectorSubcoreMesh(core_axis_name="c", subcore_axis_name="s")

    cp = pltpu.CompilerParams()
    if "needs_layout_passes" in pltpu.CompilerParams.__dataclass_fields__:
        cp = dataclasses.replace(cp, needs_layout_passes=False)

    @functools.partial(
        pl.kernel,
        out_type=jax.ShapeDtypeStruct((batch, dim), x.dtype),
        mesh=mesh,
        scratch_types=[pltpu.VMEM((dim,), jnp.int32)],
        compiler_params=cp,
    )
    def permute_kernel(x_hbm, perm_hbm, out_hbm, perm_v):
        pltpu.sync_copy(perm_hbm, perm_v)

        def body(in_v, out_v):
            @pl.loop(0, dim // LANES)
            def _(j):
                col = pl.ds(j * LANES, LANES)
                idx = perm_v[col]
                for r in range(ROWS_PER_BLOCK):
                    row = jnp.full((LANES,), r, jnp.int32)
                    out_v[r, col] = plsc.load_gather(in_v, [row, idx])

        pltpu.emit_pipeline(
            body,
            grid=(batch // ROWS_PER_BLOCK,),
            in_specs=[pl.BlockSpec((ROWS_PER_BLOCK, dim), lambda i: (i, 0))],
            out_specs=[pl.BlockSpec((ROWS_PER_BLOCK, dim), lambda i: (i, 0))],
            core_axis_name=("c", "s"),
            dimension_semantics=(pltpu.PARALLEL,),
        )(x_hbm, out_hbm)

    return permute_kernel(x, perm)


# speedup vs baseline: 3.9808x; 3.9808x over previous
import dataclasses
import functools

import jax
import jax.numpy as jnp
from jax.experimental import pallas as pl
from jax.experimental.pallas import tpu as pltpu
from jax.experimental.pallas import tpu_sc as plsc

LANES = 16
ROWS_PER_BLOCK = 4


def kernel(x, permutation):
    batch, dim = x.shape
    perm = permutation.astype(jnp.int32)
    mesh = plsc.VectorSubcoreMesh(core_axis_name="c", subcore_axis_name="s")

    cp = pltpu.CompilerParams()
    if "needs_layout_passes" in pltpu.CompilerParams.__dataclass_fields__:
        cp = dataclasses.replace(cp, needs_layout_passes=False)

    @functools.partial(
        pl.kernel,
        out_type=jax.ShapeDtypeStruct((batch, dim), x.dtype),
        mesh=mesh,
        scratch_types=[pltpu.VMEM((dim,), jnp.int32)],
        compiler_params=cp,
    )
    def permute_kernel(x_hbm, perm_hbm, out_hbm, perm_v):
        pltpu.sync_copy(perm_hbm, perm_v)

        def body(in_v, out_v):
            @plsc.parallel_loop(0, dim, step=LANES, unroll=8)
            def _(c):
                col = pl.ds(c, LANES)
                idx = perm_v[col]
                for r in range(ROWS_PER_BLOCK):
                    row = jnp.full((LANES,), r, jnp.int32)
                    out_v[r, col] = plsc.load_gather(in_v, [row, idx])

        pltpu.emit_pipeline(
            body,
            grid=(batch // ROWS_PER_BLOCK,),
            in_specs=[pl.BlockSpec((ROWS_PER_BLOCK, dim), lambda i: (i, 0))],
            out_specs=[pl.BlockSpec((ROWS_PER_BLOCK, dim), lambda i: (i, 0))],
            core_axis_name=("c", "s"),
            dimension_semantics=(pltpu.PARALLEL,),
        )(x_hbm, out_hbm)

    return permute_kernel(x, perm)
